# Initial kernel scaffold; baseline (speedup 1.0000x reference)
import jax
import jax.numpy as jnp
from jax.experimental import pallas as pl

_NS = 32


def kernel(xyz, new_xyz, features):
    diff = new_xyz[:, :, None, :] - xyz[:, None, :, :]
    d2 = jnp.sum(diff * diff, axis=-1)
    _, idx = jax.lax.top_k(-d2, _NS)
    xyz_t = jnp.transpose(xyz, (0, 2, 1))
    grouped_xyz = jax.vmap(lambda x, i: x[:, i])(xyz_t, idx)
    grouped_xyz = grouped_xyz - jnp.transpose(new_xyz, (0, 2, 1))[..., None]
    grouped_features = jax.vmap(lambda f, i: f[:, i])(features, idx)
    return jnp.concatenate([grouped_xyz, grouped_features], axis=1)


# speedup vs baseline: 5.0673x; 5.0673x over previous
import functools

import jax
import jax.numpy as jnp
from jax import lax
from jax.experimental import pallas as pl
from jax.experimental.pallas import tpu as pltpu
from jax.experimental.pallas import tpu_sc as plsc

NS = 32
CAP = 128
FLUSH_AT = CAP - 16
L = 16
ROW = 80
GQ = 4


def _round_bf16(x):
    u = lax.bitcast_convert_type(x, jnp.uint32)
    r = u + jnp.uint32(0x7FFF) + ((u >> 16) & jnp.uint32(1))
    r = r & jnp.uint32(0xFFFF0000)
    return lax.bitcast_convert_type(r, jnp.float32)


def _merge2(ak, av, bk, bv):
    rbk = lax.rev(bk, (0,))
    rbv = lax.rev(bv, (0,))
    msel = (ak < rbk) | ((ak == rbk) & (av <= rbv))
    sk = jnp.where(msel, ak, rbk)
    sv = jnp.where(msel, av, rbv)
    lk = jnp.where(msel, rbk, ak)
    lv = jnp.where(msel, rbv, av)
    sk, sv = plsc.sort_key_val(sk, sv)
    lk, lv = plsc.sort_key_val(lk, lv)
    return sk, sv, lk, lv


def _sc_body(nq_tile, n, m, pbx_h, pby_h, pbz_h, xx_h, qmeta_h, centers_h,
             table_h, out_h, px_v, py_v, pz_v, xx_v, qmeta_v, centers_v,
             cand_k, cand_v, idx_v, rows_v, sem):
    nc = 2
    wid = lax.axis_index("s") * nc + lax.axis_index("c")
    batch = wid // L
    qoff = (wid % L) * nq_tile
    gq = batch * m + qoff

    pltpu.sync_copy(pbx_h.at[batch], px_v)
    pltpu.sync_copy(pby_h.at[batch], py_v)
    pltpu.sync_copy(pbz_h.at[batch], pz_v)
    pltpu.sync_copy(xx_h.at[batch], xx_v)
    pltpu.sync_copy(qmeta_h.at[pl.ds(gq * L, nq_tile * L)], qmeta_v)
    pltpu.sync_copy(centers_h.at[pl.ds(gq * L, nq_tile * L)], centers_v)

    inf16 = jnp.full((L,), jnp.inf, jnp.float32)
    neg16 = jnp.full((L,), -1, jnp.int32)
    iota16 = lax.iota(jnp.int32, L)
    for j in range(CAP // L):
        cand_k[pl.ds(j * L, L)] = inf16
        cand_v[pl.ds(j * L, L)] = neg16

    nstep = n // L

    def flush(lo_k, lo_v, hi_k, hi_v):
        for j in range(CAP // L):
            ck = cand_k[pl.ds(j * L, L)]
            cv = cand_v[pl.ds(j * L, L)]
            ck, cv = plsc.sort_key_val(ck, cv)
            s1k, s1v, l1k, l1v = _merge2(lo_k, lo_v, ck, cv)
            s2k, s2v, _, _ = _merge2(l1k, l1v, hi_k, hi_v)
            lo_k, lo_v, hi_k, hi_v = s1k, s1v, s2k, s2v
            cand_k[pl.ds(j * L, L)] = inf16
        return lo_k, lo_v, hi_k, hi_v

    def per_query(q, _):
        qrow = qmeta_v[pl.ds(q * L, L)]
        qx = jnp.broadcast_to(qrow[0], (L,))
        qy = jnp.broadcast_to(qrow[1], (L,))
        qz = jnp.broadcast_to(qrow[2], (L,))
        qqv = jnp.broadcast_to(qrow[3], (L,))

        def step(i, carry):
            lo_k, lo_v, hi_k, hi_v, thr, cnt = carry
            base = i * L
            px = px_v[pl.ds(base, L)]
            py = py_v[pl.ds(base, L)]
            pz = pz_v[pl.ds(base, L)]
            xxv = xx_v[pl.ds(base, L)]
            inner = (qx * px + qy * py) + qz * pz
            d2 = (qqv - 2.0 * inner) + xxv
            pred = d2 <= thr
            c = jnp.sum(jnp.where(pred, jnp.int32(1), jnp.int32(0)))

            def append(args):
                lo_k, lo_v, hi_k, hi_v, thr, cnt = args
                iv = iota16 + jnp.broadcast_to(base, (L,))
                plsc.store_compressed(cand_k.at[pl.ds(cnt, L)], d2,
                                      mask=pred)
                plsc.store_compressed(cand_v.at[pl.ds(cnt, L)], iv,
                                      mask=pred)
                cnt = cnt + c

                def do_flush(args):
                    lo_k, lo_v, hi_k, hi_v, thr = args
                    lo_k, lo_v, hi_k, hi_v = flush(lo_k, lo_v, hi_k, hi_v)
                    thr = jnp.broadcast_to(hi_k[L - 1], (L,))
                    return lo_k, lo_v, hi_k, hi_v, thr

                lo_k, lo_v, hi_k, hi_v, thr = lax.cond(
                    cnt >= FLUSH_AT, do_flush,
                    lambda a: a, (lo_k, lo_v, hi_k, hi_v, thr))
                cnt = jnp.where(cnt >= FLUSH_AT, 0, cnt)
                return lo_k, lo_v, hi_k, hi_v, thr, cnt

            return lax.cond(c > 0, append, lambda a: a,
                            (lo_k, lo_v, hi_k, hi_v, thr, cnt))

        init = (inf16, neg16, inf16, neg16, inf16, jnp.int32(0))
        lo_k, lo_v, hi_k, hi_v, thr, cnt = lax.fori_loop(
            0, nstep, step, init)
        lo_k, lo_v, hi_k, hi_v = flush(lo_k, lo_v, hi_k, hi_v)
        off = jnp.broadcast_to(batch * n, (L,))
        idx_v[pl.ds(q * NS, L)] = lo_v + off
        idx_v[pl.ds(q * NS + L, L)] = hi_v + off
        return 0

    lax.fori_loop(0, nq_tile, per_query, 0)

    def per_group(g, _):
        pltpu.async_copy(table_h.at[idx_v.at[pl.ds(g * GQ * NS, GQ * NS)]],
                         rows_v, sem).wait()
        for j in range(GQ):
            cbase = (g * GQ + j) * L
            cvec = centers_v[pl.ds(cbase, L)]
            for r in range(NS):
                row = j * NS + r
                rows_v[row, pl.ds(0, L)] = rows_v[row, pl.ds(0, L)] - cvec
        out_base = (gq + g * GQ) * NS
        pltpu.sync_copy(rows_v, out_h.at[pl.ds(out_base, GQ * NS)])
        return 0

    lax.fori_loop(0, nq_tile // GQ, per_group, 0)


def kernel(xyz, new_xyz, features):
    b, n, _ = xyz.shape
    m = new_xyz.shape[1]
    c = features.shape[1]
    nw = 32
    nq_tile = (b * m) // nw

    xb = _round_bf16(xyz)
    qb = _round_bf16(new_xyz)
    xx = jnp.sum(xyz * xyz, axis=-1)
    qq = jnp.sum(new_xyz * new_xyz, axis=-1)

    pbx, pby, pbz = xb[..., 0], xb[..., 1], xb[..., 2]
    qmeta = jnp.concatenate(
        [jnp.stack([qb[..., 0], qb[..., 1], qb[..., 2], qq], axis=-1),
         jnp.zeros((b, m, L - 4), jnp.float32)], axis=-1).reshape(b * m * L)
    centers = jnp.concatenate(
        [new_xyz, jnp.zeros((b, m, L - 3), jnp.float32)],
        axis=-1).reshape(b * m * L)
    table = jnp.concatenate(
        [xyz, jnp.transpose(features, (0, 2, 1)),
         jnp.zeros((b, n, ROW - 3 - c), jnp.float32)],
        axis=-1).reshape(b * n, ROW)

    mesh = plsc.VectorSubcoreMesh(core_axis_name="c", subcore_axis_name="s")
    grouped_flat = pl.kernel(
        functools.partial(_sc_body, nq_tile, n, m),
        out_type=jax.ShapeDtypeStruct((b * m * NS, ROW), jnp.float32),
        mesh=mesh,
        scratch_types=[
            pltpu.VMEM((n,), jnp.float32),
            pltpu.VMEM((n,), jnp.float32),
            pltpu.VMEM((n,), jnp.float32),
            pltpu.VMEM((n,), jnp.float32),
            pltpu.VMEM((nq_tile * L,), jnp.float32),
            pltpu.VMEM((nq_tile * L,), jnp.float32),
            pltpu.VMEM((CAP,), jnp.float32),
            pltpu.VMEM((CAP,), jnp.int32),
            pltpu.VMEM((nq_tile * NS,), jnp.int32),
            pltpu.VMEM((GQ * NS, ROW), jnp.float32),
            pltpu.SemaphoreType.DMA,
        ],
        compiler_params=pltpu.CompilerParams(needs_layout_passes=False, use_tc_tiling_on_sc=False),
    )(pbx, pby, pbz, xx, qmeta, centers, table)

    grouped = grouped_flat.reshape(b, m, NS, ROW)
    out = jnp.transpose(grouped, (0, 3, 1, 2))[:, :3 + c]
    return out


# speedup vs baseline: 12.7785x; 2.5217x over previous
import functools

import jax
import jax.numpy as jnp
from jax import lax
from jax.experimental import pallas as pl
from jax.experimental.pallas import tpu as pltpu
from jax.experimental.pallas import tpu_sc as plsc

NS = 32
CAP = 192
SB = 4
FLUSH_AT = CAP - SB * 16
L = 16
ROW = 80
GQ = 4


def _round_bf16(x):
    u = lax.bitcast_convert_type(x, jnp.uint32)
    r = u + jnp.uint32(0x7FFF) + ((u >> 16) & jnp.uint32(1))
    r = r & jnp.uint32(0xFFFF0000)
    return lax.bitcast_convert_type(r, jnp.float32)


def _merge2(ak, av, bk, bv):
    rbk = lax.rev(bk, (0,))
    rbv = lax.rev(bv, (0,))
    msel = (ak < rbk) | ((ak == rbk) & (av <= rbv))
    sk = jnp.where(msel, ak, rbk)
    sv = jnp.where(msel, av, rbv)
    lk = jnp.where(msel, rbk, ak)
    lv = jnp.where(msel, rbv, av)
    sk, sv = plsc.sort_key_val(sk, sv)
    lk, lv = plsc.sort_key_val(lk, lv)
    return sk, sv, lk, lv


def _sc_body(nq_tile, n, m, pbx_h, pby_h, pbz_h, xx_h, qmeta_h, centers_h,
             table_h, out_h, px_v, py_v, pz_v, xx_v, qmeta_v, centers_v,
             cand_k, cand_v, idx_v, rows_v, sem):
    nc = 2
    wid = lax.axis_index("s") * nc + lax.axis_index("c")
    batch = wid // L
    qoff = (wid % L) * nq_tile
    gq = batch * m + qoff

    pltpu.sync_copy(pbx_h.at[batch], px_v)
    pltpu.sync_copy(pby_h.at[batch], py_v)
    pltpu.sync_copy(pbz_h.at[batch], pz_v)
    pltpu.sync_copy(xx_h.at[batch], xx_v)
    pltpu.sync_copy(qmeta_h.at[pl.ds(gq * L, nq_tile * L)], qmeta_v)
    pltpu.sync_copy(centers_h.at[pl.ds(gq * L, nq_tile * L)], centers_v)

    inf16 = jnp.full((L,), jnp.inf, jnp.float32)
    neg16 = jnp.full((L,), -1, jnp.int32)
    iota16 = lax.iota(jnp.int32, L)
    for j in range(CAP // L):
        cand_k[pl.ds(j * L, L)] = inf16
        cand_v[pl.ds(j * L, L)] = neg16

    nstep = n // L

    def flush(lo_k, lo_v, hi_k, hi_v):
        for j in range(CAP // L):
            ck = cand_k[pl.ds(j * L, L)]
            cv = cand_v[pl.ds(j * L, L)]
            ck, cv = plsc.sort_key_val(ck, cv)
            s1k, s1v, l1k, l1v = _merge2(lo_k, lo_v, ck, cv)
            s2k, s2v, _, _ = _merge2(l1k, l1v, hi_k, hi_v)
            lo_k, lo_v, hi_k, hi_v = s1k, s1v, s2k, s2v
            cand_k[pl.ds(j * L, L)] = inf16
        return lo_k, lo_v, hi_k, hi_v

    def per_query(q, _):
        qrow = qmeta_v[pl.ds(q * L, L)]
        qx = jnp.broadcast_to(qrow[0], (L,))
        qy = jnp.broadcast_to(qrow[1], (L,))
        qz = jnp.broadcast_to(qrow[2], (L,))
        qqv = jnp.broadcast_to(qrow[3], (L,))

        def step(g, carry):
            lo_k, lo_v, hi_k, hi_v, thr, cnt = carry
            gbase = g * (SB * L)
            d2s, preds = [], []
            for j in range(SB):
                base = gbase + j * L
                px = px_v[pl.ds(base, L)]
                py = py_v[pl.ds(base, L)]
                pz = pz_v[pl.ds(base, L)]
                xxv = xx_v[pl.ds(base, L)]
                inner = (qx * px + qy * py) + qz * pz
                d2 = (qqv - 2.0 * inner) + xxv
                d2s.append(d2)
                preds.append(d2 <= thr)
            many = (preds[0] | preds[1]) | (preds[2] | preds[3])
            anyc = plsc.all_reduce_population_count(many)[0]

            def append(args):
                lo_k, lo_v, hi_k, hi_v, thr, cnt = args
                for j in range(SB):
                    iv = iota16 + jnp.broadcast_to(gbase + j * L, (L,))
                    cj = plsc.all_reduce_population_count(preds[j])[0]
                    plsc.store_compressed(cand_k.at[pl.ds(cnt, L)], d2s[j],
                                          mask=preds[j])
                    plsc.store_compressed(cand_v.at[pl.ds(cnt, L)], iv,
                                          mask=preds[j])
                    cnt = cnt + cj

                def do_flush(args):
                    lo_k, lo_v, hi_k, hi_v, thr = args
                    lo_k, lo_v, hi_k, hi_v = flush(lo_k, lo_v, hi_k, hi_v)
                    thr = jnp.broadcast_to(hi_k[L - 1], (L,))
                    return lo_k, lo_v, hi_k, hi_v, thr

                lo_k, lo_v, hi_k, hi_v, thr = lax.cond(
                    cnt >= FLUSH_AT, do_flush,
                    lambda a: a, (lo_k, lo_v, hi_k, hi_v, thr))
                cnt = jnp.where(cnt >= FLUSH_AT, 0, cnt)
                return lo_k, lo_v, hi_k, hi_v, thr, cnt

            return lax.cond(anyc > 0, append, lambda a: a,
                            (lo_k, lo_v, hi_k, hi_v, thr, cnt))

        init = (inf16, neg16, inf16, neg16, inf16, jnp.int32(0))
        lo_k, lo_v, hi_k, hi_v, thr, cnt = lax.fori_loop(
            0, nstep // SB, step, init)
        lo_k, lo_v, hi_k, hi_v = flush(lo_k, lo_v, hi_k, hi_v)
        off = jnp.broadcast_to(batch * n, (L,))
        idx_v[pl.ds(q * NS, L)] = lo_v + off
        idx_v[pl.ds(q * NS + L, L)] = hi_v + off
        return 0

    lax.fori_loop(0, nq_tile, per_query, 0)

    def per_group(g, _):
        pltpu.async_copy(table_h.at[idx_v.at[pl.ds(g * GQ * NS, GQ * NS)]],
                         rows_v, sem).wait()
        for j in range(GQ):
            cbase = (g * GQ + j) * L
            cvec = centers_v[pl.ds(cbase, L)]
            for r in range(NS):
                row = j * NS + r
                rows_v[row, pl.ds(0, L)] = rows_v[row, pl.ds(0, L)] - cvec
        out_base = (gq + g * GQ) * NS
        pltpu.sync_copy(rows_v, out_h.at[pl.ds(out_base, GQ * NS)])
        return 0

    lax.fori_loop(0, nq_tile // GQ, per_group, 0)


def kernel(xyz, new_xyz, features):
    b, n, _ = xyz.shape
    m = new_xyz.shape[1]
    c = features.shape[1]
    nw = 32
    nq_tile = (b * m) // nw

    xb = _round_bf16(xyz)
    qb = _round_bf16(new_xyz)
    xx = jnp.sum(xyz * xyz, axis=-1)
    qq = jnp.sum(new_xyz * new_xyz, axis=-1)

    pbx, pby, pbz = xb[..., 0], xb[..., 1], xb[..., 2]
    qmeta = jnp.concatenate(
        [jnp.stack([qb[..., 0], qb[..., 1], qb[..., 2], qq], axis=-1),
         jnp.zeros((b, m, L - 4), jnp.float32)], axis=-1).reshape(b * m * L)
    centers = jnp.concatenate(
        [new_xyz, jnp.zeros((b, m, L - 3), jnp.float32)],
        axis=-1).reshape(b * m * L)
    table = jnp.concatenate(
        [xyz, jnp.transpose(features, (0, 2, 1)),
         jnp.zeros((b, n, ROW - 3 - c), jnp.float32)],
        axis=-1).reshape(b * n, ROW)

    mesh = plsc.VectorSubcoreMesh(core_axis_name="c", subcore_axis_name="s")
    grouped_flat = pl.kernel(
        functools.partial(_sc_body, nq_tile, n, m),
        out_type=jax.ShapeDtypeStruct((b * m * NS, ROW), jnp.float32),
        mesh=mesh,
        scratch_types=[
            pltpu.VMEM((n,), jnp.float32),
            pltpu.VMEM((n,), jnp.float32),
            pltpu.VMEM((n,), jnp.float32),
            pltpu.VMEM((n,), jnp.float32),
            pltpu.VMEM((nq_tile * L,), jnp.float32),
            pltpu.VMEM((nq_tile * L,), jnp.float32),
            pltpu.VMEM((CAP,), jnp.float32),
            pltpu.VMEM((CAP,), jnp.int32),
            pltpu.VMEM((nq_tile * NS,), jnp.int32),
            pltpu.VMEM((GQ * NS, ROW), jnp.float32),
            pltpu.SemaphoreType.DMA,
        ],
        compiler_params=pltpu.CompilerParams(needs_layout_passes=False, use_tc_tiling_on_sc=False),
    )(pbx, pby, pbz, xx, qmeta, centers, table)

    grouped = grouped_flat.reshape(b, m, NS, ROW)
    out = jnp.transpose(grouped, (0, 3, 1, 2))[:, :3 + c]
    return out
